# Initial kernel scaffold; baseline (speedup 1.0000x reference)
import jax
import jax.numpy as jnp
from jax.experimental import pallas as pl


def kernel(seq_len, relative_embeddings):
    raise NotImplementedError("write your pallas kernel here")


# speedup vs baseline: 9.8576x; 9.8576x over previous
import functools

import jax
import jax.numpy as jnp
from jax import lax
from jax.experimental import pallas as pl
from jax.experimental.pallas import tpu as pltpu
from jax.experimental.pallas import tpu_sc as plsc


def kernel(seq_len, relative_embeddings):
    del seq_len
    two_max_len, embed = relative_embeddings.shape
    s = two_max_len // 2
    row_words = s * embed
    table_words = two_max_len * embed

    info = plsc.get_sparse_core_info()
    num_workers = info.num_cores * info.num_subcores
    rows_per_w = s // num_workers

    mesh = plsc.VectorSubcoreMesh(core_axis_name="c", subcore_axis_name="s")

    @functools.partial(
        pl.kernel,
        mesh=mesh,
        out_type=jax.ShapeDtypeStruct((s * row_words,), jnp.float32),
        scratch_types=[pltpu.VMEM((table_words,), jnp.float32)],
    )
    def toeplitz_rows(table_hbm, out_hbm, table_v):
        wid = lax.axis_index("s") * info.num_cores + lax.axis_index("c")
        pltpu.sync_copy(table_hbm, table_v)
        base = wid * rows_per_w

        def body(r, carry):
            i = base + r
            src_start = (s - i) * embed
            pltpu.sync_copy(
                table_v.at[pl.ds(src_start, row_words)],
                out_hbm.at[pl.ds(i * row_words, row_words)],
            )
            return carry

        lax.fori_loop(0, rows_per_w, body, 0)

    flat = toeplitz_rows(relative_embeddings.reshape(table_words))
    return flat.reshape(s, s, embed)
